# baseline (speedup 1.0000x reference)
import functools

import jax
import jax.numpy as jnp
from jax import lax
from jax.experimental import pallas as pl
from jax.experimental.pallas import tpu as pltpu
from jax.experimental.pallas import tpu_sc as plsc

_N_NODES = 2000
_N_EDGES = 6000
_BATCH = 4
_DIMS = [8, 32, 16, 8, 2]
_HOPS = [3, 3, 3]
_SLOPE = 0.01
_BN_EPS = 1e-5

_FC_IN = _DIMS[-2] * _N_NODES
_FC_OUT = _DIMS[-1] * _N_NODES

_KT = 3200
_NT = 512
_FC_OUT_PAD = 4096


def _fc_body(x_ref, w_ref, b_ref, m_ref, o_ref):
    k = pl.program_id(0)
    n = pl.program_id(1)
    nk = pl.num_programs(0)
    nn = pl.num_programs(1)

    @pl.when(k == 0)
    def _init():
        o_ref[:, pl.ds(n * _NT, _NT)] = jnp.zeros((_BATCH, _NT), jnp.float32)

    o_ref[:, pl.ds(n * _NT, _NT)] += jnp.dot(
        x_ref[...], w_ref[...], preferred_element_type=jnp.float32)

    @pl.when(jnp.logical_and(k == nk - 1, n == nn - 1))
    def _fini():
        o_ref[:, :_FC_OUT] = ((o_ref[:, :_FC_OUT] + b_ref[...])
                              * m_ref[...])


@jax.jit
def _fc_pallas(x2d, fc_w, fc_b, mask_flat):
    grid = (_FC_IN // _KT, _FC_OUT_PAD // _NT)
    y = pl.pallas_call(
        _fc_body,
        grid=grid,
        in_specs=[
            pl.BlockSpec((_BATCH, _KT), lambda k, n: (0, k)),
            pl.BlockSpec((_KT, _NT), lambda k, n: (k, n)),
            pl.BlockSpec((1, _FC_OUT), lambda k, n: (0, 0)),
            pl.BlockSpec((1, _FC_OUT), lambda k, n: (0, 0)),
        ],
        out_specs=pl.BlockSpec((_BATCH, _FC_OUT_PAD), lambda k, n: (0, 0)),
        out_shape=jax.ShapeDtypeStruct((_BATCH, _FC_OUT_PAD), jnp.float32),
        compiler_params=pltpu.CompilerParams(
            dimension_semantics=("arbitrary", "arbitrary"),
        ),
    )(x2d, fc_w, fc_b[None, :], mask_flat[None, :])
    return y[:, :_FC_OUT]


_ROWS_PER_TILE = 64
_A_ROWS = 32 * _ROWS_PER_TILE
_ECHUNK = 400


def _build_a0_sc():
    mesh = plsc.VectorSubcoreMesh(core_axis_name="c", subcore_axis_name="s")

    blk_words = _ROWS_PER_TILE * _N_NODES
    n_chunks = _N_EDGES // _ECHUNK

    @functools.partial(
        pl.kernel,
        out_type=jax.ShapeDtypeStruct((_A_ROWS * _N_NODES,), jnp.float32),
        mesh=mesh,
        compiler_params=pltpu.CompilerParams(needs_layout_passes=False),
        scratch_types=[
            pltpu.VMEM((blk_words,), jnp.float32),
            pltpu.VMEM((_ECHUNK,), jnp.int32),
            pltpu.VMEM((_ECHUNK,), jnp.int32),
            pltpu.VMEM((_ECHUNK,), jnp.float32),
            pltpu.VMEM((_ECHUNK,), jnp.int32),
            pltpu.VMEM((_ECHUNK,), jnp.int32),
            pltpu.VMEM((_ECHUNK,), jnp.float32),
            pltpu.SemaphoreType.DMA,
            pltpu.SemaphoreType.DMA,
            pltpu.SemaphoreType.DMA,
            pltpu.SemaphoreType.DMA,
            pltpu.SemaphoreType.DMA,
            pltpu.SemaphoreType.DMA,
        ],
    )
    def a0_kernel(src_hbm, dst_hbm, ew_hbm, a_hbm, ablk,
                  srcv0, dstv0, ewv0, srcv1, dstv1, ewv1, *sems):
        wid = lax.axis_index("s") * 2 + lax.axis_index("c")
        base = wid * _ROWS_PER_TILE
        bufs = ((srcv0, dstv0, ewv0), (srcv1, dstv1, ewv1))

        def issue(c):
            buf = c % 2
            off = c * _ECHUNK
            sv, dv, ev = bufs[buf]
            return (
                pltpu.async_copy(src_hbm.at[pl.ds(off, _ECHUNK)],
                                 sv, sems[buf * 3 + 0]),
                pltpu.async_copy(dst_hbm.at[pl.ds(off, _ECHUNK)],
                                 dv, sems[buf * 3 + 1]),
                pltpu.async_copy(ew_hbm.at[pl.ds(off, _ECHUNK)],
                                 ev, sems[buf * 3 + 2]),
            )

        pending = issue(0)

        def zero_vec(c, _):
            for j in range(8):
                ablk[pl.ds(c * 128 + j * 16, 16)] = jnp.zeros((16,),
                                                              jnp.float32)
            return 0

        lax.fori_loop(0, blk_words // 128, zero_vec, 0)

        lanes = lax.iota(jnp.int32, 16)

        for chunk in range(n_chunks):
            sv, dv, ev = bufs[chunk % 2]
            for h in pending:
                h.wait()
            if chunk + 1 < n_chunks:
                pending = issue(chunk + 1)

            def scan_group(g, _):
                goff = g * 16
                src16 = sv[pl.ds(goff, 16)]
                r16 = dv[pl.ds(goff, 16)] - base
                ew16 = ev[pl.ds(goff, 16)]
                m_in = jnp.logical_and(r16 >= 0, r16 < _ROWS_PER_TILE)
                cnt = plsc.all_reduce_population_count(m_in)

                @pl.when(cnt[0] > 0)
                def _():
                    flat16 = r16 * _N_NODES + src16

                    def peel(_, m32):
                        m = m32 > 0
                        e = plsc.all_reduce_ffs(m)
                        m_e = jnp.logical_and(m, lanes == e)
                        old = plsc.load_gather(ablk, [flat16], mask=m_e)
                        plsc.store_scatter(ablk, [flat16], old + ew16,
                                           mask=m_e)
                        return jnp.where(m_e, 0, m32)

                    lax.fori_loop(0, cnt[0], peel, m_in.astype(jnp.int32))
                return 0

            lax.fori_loop(0, _ECHUNK // 16, scan_group, 0)

        pltpu.sync_copy(ablk, a_hbm.at[pl.ds(base * _N_NODES, blk_words)])

    return a0_kernel


_A0_SC = _build_a0_sc()


def _leaky(v):
    return jnp.where(v >= 0, v, _SLOPE * v)


def _bn_rows(o, gamma_col, beta_col):
    mu = jnp.mean(o, axis=1, keepdims=True)
    var = jnp.mean((o - mu) ** 2, axis=1, keepdims=True)
    return (o - mu) * jax.lax.rsqrt(var + _BN_EPS) * gamma_col + beta_col


def _graph_body(a_ref, x_ref, w_refs, b_refs, g_refs, t_refs, out_ref):
    a = a_ref[...]
    deg = jnp.sum(a, axis=1, keepdims=True)
    dinv = jnp.where(deg > 0, jax.lax.rsqrt(deg), 0.0)
    dinv_n = dinv[:_N_NODES]

    def prop(h):
        u = jnp.dot(a, dinv_n * h, preferred_element_type=jnp.float32)
        return (dinv * u)[:_N_NODES]

    s0 = jnp.concatenate([x_ref[b] for b in range(_BATCH)], axis=1)
    s1 = prop(s0)
    s2 = prop(s1)
    s3 = prop(s2)
    s = jnp.concatenate([s0, s1, s2, s3], axis=1)
    o = jnp.dot(s, w_refs[0][...], preferred_element_type=jnp.float32)
    o = o + b_refs[0][...]
    o = _leaky(_bn_rows(o, g_refs[0][...], t_refs[0][...]))

    g = jnp.dot(o, w_refs[1][...], preferred_element_type=jnp.float32)
    acc = g[:, 192:256]
    acc = g[:, 128:192] + prop(acc)
    acc = g[:, 64:128] + prop(acc)
    acc = g[:, 0:64] + prop(acc)
    o = acc + b_refs[1][...]
    o = _leaky(_bn_rows(o, g_refs[1][...], t_refs[1][...]))

    g = jnp.dot(o, w_refs[2][...], preferred_element_type=jnp.float32)
    acc = g[:, 96:128]
    acc = g[:, 64:96] + prop(acc)
    acc = g[:, 32:64] + prop(acc)
    acc = g[:, 0:32] + prop(acc)
    o = acc + b_refs[2][...]
    o = _leaky(_bn_rows(o, g_refs[2][...], t_refs[2][...]))
    out_ref[...] = o


def _graph_body_flat(a_ref, x_ref, m1, m2, m3, cb1, cb2, cb3,
                     bg1, bt1, bg2, bt2, bg3, bt3, out_ref):
    _graph_body(a_ref, x_ref, [m1, m2, m3], [cb1, cb2, cb3],
                [bg1, bg2, bg3], [bt1, bt2, bt3], out_ref)


@jax.jit
def _graph_pallas(a0, x, conv_weights, conv_biases, bn_gamma, bn_beta):
    eye_b = jnp.eye(_BATCH, dtype=jnp.float32)
    m1 = jnp.concatenate(
        [jnp.kron(eye_b, w) for w in conv_weights[0]], axis=0)
    m2 = jnp.concatenate(
        [jnp.kron(eye_b, w) for w in conv_weights[1]], axis=1)
    m3 = jnp.concatenate(
        [jnp.kron(eye_b, w) for w in conv_weights[2]], axis=1)
    cbs = [jnp.tile(b, _BATCH)[None, :] for b in conv_biases]
    cols = []
    for layer in range(3):
        cols += [bn_gamma[layer][:, None], bn_beta[layer][:, None]]
    return pl.pallas_call(
        _graph_body_flat,
        out_shape=jax.ShapeDtypeStruct((_N_NODES, _BATCH * _DIMS[3]),
                                       jnp.float32),
    )(a0, x, m1, m2, m3, *cbs, *cols)


def kernel(x, edge_index, edge_weights, feature_mask, conv_weights,
           conv_biases, bn_gamma, bn_beta, fc_w, fc_b):
    a0 = _A0_SC(edge_index[0], edge_index[1], edge_weights).reshape(
        _A_ROWS, _N_NODES)
    out3 = _graph_pallas(a0, x, conv_weights, conv_biases, bn_gamma, bn_beta)
    x2d = out3.reshape(_N_NODES, _BATCH, _DIMS[3]).transpose(1, 0, 2).reshape(
        _BATCH, _FC_IN)
    y = _fc_pallas(x2d, fc_w, fc_b, feature_mask.reshape(-1))
    return y.reshape(_BATCH, _N_NODES, _DIMS[-1])
